# Initial kernel scaffold; baseline (speedup 1.0000x reference)
import jax
import jax.numpy as jnp
from jax.experimental import pallas as pl


def kernel(x, edge_index, edge_label_index, W1_l, W1_r, b1, W2_l, W2_r, b2):
    raise NotImplementedError("write your pallas kernel here")


# speedup vs baseline: 4.3890x; 4.3890x over previous
import functools

import jax
import jax.numpy as jnp
from jax import lax
from jax.experimental import pallas as pl
from jax.experimental.pallas import tpu as pltpu
from jax.experimental.pallas import tpu_sc as plsc

N_NODES = 10000
IN_DIM = 128
HID_DIM = 128
OUT_DIM = 64
N_EDGES = 320000
N_LABEL = 100000

NC, NS = 2, 16
NW = NC * NS
CHUNK = 128

N_PAD = 10240
ROWS_PER_SUB = N_PAD // NS

AGG_W = 144

E_CPW = 79
E_PAD = NW * E_CPW * CHUNK

L_CPW = 25
L_PAD = NW * L_CPW * CHUNK

_MESH = plsc.VectorSubcoreMesh(core_axis_name="c", subcore_axis_name="s")


def _make_segsum(width, chunks_per_worker):
  @functools.partial(
      pl.kernel,
      out_type=jax.ShapeDtypeStruct((NC, N_PAD, width), jnp.float32),
      mesh=_MESH,
      compiler_params=pltpu.CompilerParams(use_tc_tiling_on_sc=False, needs_layout_passes=False),
      scratch_types=[
          pltpu.VMEM((chunks_per_worker, CHUNK), jnp.int32),
          pltpu.VMEM((chunks_per_worker, CHUNK), jnp.int32),
          pltpu.VMEM((CHUNK, width), jnp.float32),
          pltpu.VMEM_SHARED((N_PAD, width), jnp.float32),
          pltpu.SemaphoreType.DMA,
      ],
  )
  def segsum(tab_hbm, src_hbm, dst_hbm, zeros_hbm, out_hbm,
             src_v, dst_v, rows_v, acc_sh, sem):
    cid = lax.axis_index("c")
    sid = lax.axis_index("s")
    wid = cid * NS + sid
    r0 = sid * ROWS_PER_SUB
    pltpu.sync_copy(zeros_hbm.at[pl.ds(r0, ROWS_PER_SUB)],
                    acc_sh.at[pl.ds(r0, ROWS_PER_SUB)])
    plsc.subcore_barrier()
    pltpu.sync_copy(src_hbm.at[wid], src_v)
    pltpu.sync_copy(dst_hbm.at[wid], dst_v)

    def body(j, carry):
      pltpu.async_copy(tab_hbm.at[src_v.at[j]], rows_v, sem).wait()
      pltpu.sync_copy(rows_v, acc_sh.at[dst_v.at[j]], add=True)
      return carry

    lax.fori_loop(0, chunks_per_worker, body, 0)
    plsc.subcore_barrier()
    pltpu.sync_copy(acc_sh.at[pl.ds(r0, ROWS_PER_SUB)],
                    out_hbm.at[cid, pl.ds(r0, ROWS_PER_SUB)])

  return segsum


_segsum_l1 = _make_segsum(AGG_W, E_CPW)
_segsum_l2 = _make_segsum(OUT_DIM, E_CPW)


@functools.partial(
    pl.kernel,
    out_type=jax.ShapeDtypeStruct((L_PAD,), jnp.float32),
    mesh=_MESH,
    compiler_params=pltpu.CompilerParams(use_tc_tiling_on_sc=False, needs_layout_passes=False),
    scratch_types=[
        pltpu.VMEM((L_CPW, CHUNK), jnp.int32),
        pltpu.VMEM((L_CPW, CHUNK), jnp.int32),
        pltpu.VMEM((CHUNK, OUT_DIM), jnp.float32),
        pltpu.VMEM((CHUNK, OUT_DIM), jnp.float32),
        pltpu.VMEM((CHUNK,), jnp.float32),
        pltpu.SemaphoreType.DMA,
    ],
)
def _decode(z_hbm, ls_hbm, ld_hbm, out_hbm, ls_v, ld_v, zs_v, zd_v, out_v, sem):
  cid = lax.axis_index("c")
  sid = lax.axis_index("s")
  wid = cid * NS + sid
  ib = wid * L_CPW
  pltpu.sync_copy(ls_hbm.at[wid], ls_v)
  pltpu.sync_copy(ld_hbm.at[wid], ld_v)

  def chunk(j, carry):
    pltpu.async_copy(z_hbm.at[ls_v.at[j]], zs_v, sem).wait()
    pltpu.async_copy(z_hbm.at[ld_v.at[j]], zd_v, sem).wait()

    def group(g, c2):
      rows = g * 16 + lax.iota(jnp.int32, 16)
      acc = jnp.zeros((16,), jnp.float32)
      for col in range(OUT_DIM):
        cv = jnp.full((16,), col, jnp.int32)
        acc = acc + (plsc.load_gather(zs_v, [rows, cv]) *
                     plsc.load_gather(zd_v, [rows, cv]))
      out_v[pl.ds(g * 16, 16)] = acc
      return c2

    lax.fori_loop(0, CHUNK // 16, group, 0)
    pltpu.sync_copy(out_v, out_hbm.at[pl.ds((ib + j) * CHUNK, CHUNK)])
    return carry

  lax.fori_loop(0, L_CPW, chunk, 0)


def _layer1_body(aggp, xr, w1l, w1r, b1, w2l, w2r, b2, hw, hr, ic):
  a = aggp[0] + aggp[1]
  inv = 1.0 / jnp.maximum(a[:, IN_DIM], 1.0)
  mean = a[:, :IN_DIM] * inv[:, None]
  dn = (((1,), (1,)), ((), ()))
  h = (lax.dot_general(mean, w1l[...], dn, preferred_element_type=jnp.float32)
       + lax.dot_general(xr[...], w1r[...], dn, preferred_element_type=jnp.float32)
       + b1[...])
  h = jnp.maximum(h, 0.0)
  hw[...] = lax.dot_general(h, w2l[...], dn, preferred_element_type=jnp.float32)
  hr[...] = (lax.dot_general(h, w2r[...], dn, preferred_element_type=jnp.float32)
             + b2[...])
  ic[...] = inv[:, None]


def _layer1(aggp, x, W1_l, W1_r, b1, W2_l, W2_r, b2):
  BR = 1000
  grid = (N_NODES // BR,)
  return pl.pallas_call(
      lambda *refs: _layer1_body(refs[0][...], *refs[1:]),
      grid=grid,
      in_specs=[
          pl.BlockSpec((NC, BR, AGG_W), lambda i: (0, i, 0)),
          pl.BlockSpec((BR, IN_DIM), lambda i: (i, 0)),
          pl.BlockSpec((HID_DIM, IN_DIM), lambda i: (0, 0)),
          pl.BlockSpec((HID_DIM, IN_DIM), lambda i: (0, 0)),
          pl.BlockSpec((1, HID_DIM), lambda i: (0, 0)),
          pl.BlockSpec((OUT_DIM, HID_DIM), lambda i: (0, 0)),
          pl.BlockSpec((OUT_DIM, HID_DIM), lambda i: (0, 0)),
          pl.BlockSpec((1, OUT_DIM), lambda i: (0, 0)),
      ],
      out_specs=[
          pl.BlockSpec((BR, OUT_DIM), lambda i: (i, 0)),
          pl.BlockSpec((BR, OUT_DIM), lambda i: (i, 0)),
          pl.BlockSpec((BR, 1), lambda i: (i, 0)),
      ],
      out_shape=[
          jax.ShapeDtypeStruct((N_NODES, OUT_DIM), jnp.float32),
          jax.ShapeDtypeStruct((N_NODES, OUT_DIM), jnp.float32),
          jax.ShapeDtypeStruct((N_NODES, 1), jnp.float32),
      ],
  )(aggp, x, W1_l, W1_r, b1, W2_l, W2_r, b2)


def _assemble_body(aggp2, ic, hr, z):
  s = aggp2[0] + aggp2[1]
  z[...] = s * ic[...] + hr[...]


def _assemble_z(aggp2, ic, hr):
  BR = 1000
  return pl.pallas_call(
      lambda *refs: _assemble_body(refs[0][...], *refs[1:]),
      grid=(N_NODES // BR,),
      in_specs=[
          pl.BlockSpec((NC, BR, OUT_DIM), lambda i: (0, i, 0)),
          pl.BlockSpec((BR, 1), lambda i: (i, 0)),
          pl.BlockSpec((BR, OUT_DIM), lambda i: (i, 0)),
      ],
      out_specs=pl.BlockSpec((BR, OUT_DIM), lambda i: (i, 0)),
      out_shape=jax.ShapeDtypeStruct((N_NODES, OUT_DIM), jnp.float32),
  )(aggp2, ic, hr)


def kernel(x, edge_index, edge_label_index, W1_l, W1_r, b1, W2_l, W2_r, b2):
  i32 = jnp.int32
  f32 = jnp.float32
  src = edge_index[0].astype(i32)
  dst = edge_index[1].astype(i32)
  ls = edge_label_index[0].astype(i32)
  ld = edge_label_index[1].astype(i32)

  ep = E_PAD - N_EDGES
  src2 = jnp.concatenate([src, jnp.zeros((ep,), i32)]).reshape(NW, E_CPW, CHUNK)
  dst2 = jnp.concatenate([dst, jnp.full((ep,), N_PAD - 1, i32)]).reshape(
      NW, E_CPW, CHUNK)
  lp = L_PAD - N_LABEL
  ls2 = jnp.concatenate([ls, jnp.zeros((lp,), i32)]).reshape(NW, L_CPW, CHUNK)
  ld2 = jnp.concatenate([ld, jnp.zeros((lp,), i32)]).reshape(NW, L_CPW, CHUNK)

  xa = jnp.concatenate(
      [x, jnp.ones((N_NODES, 1), f32), jnp.zeros((N_NODES, AGG_W - IN_DIM - 1), f32)],
      axis=1)

  aggp1 = _segsum_l1(xa, src2, dst2, jnp.zeros((N_PAD, AGG_W), f32))
  hw, hr, ic = _layer1(aggp1, x, W1_l, W1_r, b1.reshape(1, HID_DIM),
                       W2_l, W2_r, b2.reshape(1, OUT_DIM))
  aggp2 = _segsum_l2(hw, src2, dst2, jnp.zeros((N_PAD, OUT_DIM), f32))
  z = _assemble_z(aggp2, ic, hr)
  out = _decode(z, ls2, ld2)
  return out[:N_LABEL]


# speedup vs baseline: 4.9991x; 1.1390x over previous
import functools

import jax
import jax.numpy as jnp
from jax import lax
from jax.experimental import pallas as pl
from jax.experimental.pallas import tpu as pltpu
from jax.experimental.pallas import tpu_sc as plsc

N_NODES = 10000
IN_DIM = 128
HID_DIM = 128
OUT_DIM = 64
N_EDGES = 320000
N_LABEL = 100000

NC, NS = 2, 16
NW = NC * NS
CHUNK = 128

N_PAD = 10240
ROWS_PER_SUB = N_PAD // NS

HALF_W = 72
CNT_COL = IN_DIM - HALF_W

E1_CPW = 157
E1_PAD = NS * E1_CPW * CHUNK

E2_CPW = 80
E2_PAD = NW * E2_CPW * CHUNK

L_CPW = 25
L_PAD = NW * L_CPW * CHUNK

_MESH = plsc.VectorSubcoreMesh(core_axis_name="c", subcore_axis_name="s")
_SC_PARAMS = pltpu.CompilerParams(use_tc_tiling_on_sc=False,
                                  needs_layout_passes=False)


def _zero_rows(buf, width):
  z16 = jnp.zeros((16,), jnp.float32)
  ncol = (width + 15) // 16

  def zrow(r, carry):
    for c in range(ncol):
      buf[r, pl.ds(min(c * 16, width - 16), 16)] = z16
    return carry

  lax.fori_loop(0, CHUNK, zrow, 0)


def _make_segsum(width, cpw, split_cols):
  @functools.partial(
      pl.kernel,
      out_type=jax.ShapeDtypeStruct((NC, N_PAD, width), jnp.float32),
      mesh=_MESH,
      compiler_params=_SC_PARAMS,
      scratch_types=[
          pltpu.VMEM((cpw, CHUNK), jnp.int32),
          pltpu.VMEM((cpw, CHUNK), jnp.int32),
          pltpu.VMEM((2 * CHUNK, width), jnp.float32),
          pltpu.VMEM_SHARED((N_PAD, width), jnp.float32),
          pltpu.SemaphoreType.DMA,
          pltpu.SemaphoreType.DMA,
      ],
  )
  def segsum(tab_hbm, src_hbm, dst_hbm, out_hbm,
             src_v, dst_v, rows2, acc_sh, gsem, ssem):
    cid = lax.axis_index("c")
    sid = lax.axis_index("s")
    r0 = sid * ROWS_PER_SUB
    tab = tab_hbm.at[cid] if split_cols else tab_hbm
    isel = sid if split_cols else cid * NS + sid

    _zero_rows(rows2, width)
    for t in range(ROWS_PER_SUB // CHUNK):
      pltpu.sync_copy(rows2.at[pl.ds(0, CHUNK)],
                      acc_sh.at[pl.ds(r0 + t * CHUNK, CHUNK)])
    plsc.subcore_barrier()
    pltpu.sync_copy(src_hbm.at[isel], src_v)
    pltpu.sync_copy(dst_hbm.at[isel], dst_v)

    def buf(v):
      return rows2.at[pl.ds((v % 2) * CHUNK, CHUNK)]

    def g_start(j, v):
      pltpu.async_copy(tab.at[src_v.at[j]], buf(v), gsem)

    def g_wait(v):
      pltpu.make_async_copy(tab.at[src_v.at[0]], buf(v), gsem).wait()

    def s_start(v):
      pltpu.async_copy(buf(v), acc_sh.at[dst_v.at[v]], ssem, add=True)

    def s_wait():
      pltpu.make_async_copy(buf(0), acc_sh.at[dst_v.at[0]], ssem).wait()

    g_start(0, 0)
    g_wait(0)
    s_start(0)
    g_start(1, 1)

    def body(v, carry):
      g_wait(v)
      s_wait()
      s_start(v)
      g_start(jnp.minimum(v + 1, cpw - 1), v + 1)
      return carry

    lax.fori_loop(1, cpw, body, 0)
    g_wait(cpw)
    s_wait()
    plsc.subcore_barrier()
    pltpu.sync_copy(acc_sh.at[pl.ds(r0, ROWS_PER_SUB)],
                    out_hbm.at[cid, pl.ds(r0, ROWS_PER_SUB)])

  return segsum


_segsum_l1 = _make_segsum(HALF_W, E1_CPW, split_cols=True)
_segsum_l2 = _make_segsum(OUT_DIM, E2_CPW, split_cols=False)


@functools.partial(
    pl.kernel,
    out_type=jax.ShapeDtypeStruct((L_PAD,), jnp.float32),
    mesh=_MESH,
    compiler_params=_SC_PARAMS,
    scratch_types=[
        pltpu.VMEM((L_CPW, CHUNK), jnp.int32),
        pltpu.VMEM((L_CPW, CHUNK), jnp.int32),
        pltpu.VMEM((2 * CHUNK, OUT_DIM), jnp.float32),
        pltpu.VMEM((2 * CHUNK, OUT_DIM), jnp.float32),
        pltpu.VMEM((CHUNK,), jnp.float32),
        pltpu.SemaphoreType.DMA,
        pltpu.SemaphoreType.DMA,
    ],
)
def _decode(z_hbm, ls_hbm, ld_hbm, out_hbm, ls_v, ld_v, zs2, zd2, out_v,
            ssm, dsm):
  cid = lax.axis_index("c")
  sid = lax.axis_index("s")
  wid = cid * NS + sid
  ib = wid * L_CPW
  pltpu.sync_copy(ls_hbm.at[wid], ls_v)
  pltpu.sync_copy(ld_hbm.at[wid], ld_v)

  def g_start(j, v):
    boff = (v % 2) * CHUNK
    pltpu.async_copy(z_hbm.at[ls_v.at[j]], zs2.at[pl.ds(boff, CHUNK)], ssm)
    pltpu.async_copy(z_hbm.at[ld_v.at[j]], zd2.at[pl.ds(boff, CHUNK)], dsm)

  def g_wait(v):
    boff = (v % 2) * CHUNK
    pltpu.make_async_copy(z_hbm.at[ls_v.at[0]], zs2.at[pl.ds(boff, CHUNK)],
                          ssm).wait()
    pltpu.make_async_copy(z_hbm.at[ld_v.at[0]], zd2.at[pl.ds(boff, CHUNK)],
                          dsm).wait()

  g_start(0, 0)

  def visit(v, carry):
    g_wait(v)
    g_start(jnp.minimum(v + 1, L_CPW - 1), v + 1)
    boff = (v % 2) * CHUNK

    def group(g, c2):
      rows = boff + g * 16 + lax.iota(jnp.int32, 16)
      acc = jnp.zeros((16,), jnp.float32)
      for col in range(OUT_DIM):
        cv = jnp.full((16,), col, jnp.int32)
        acc = acc + (plsc.load_gather(zs2, [rows, cv]) *
                     plsc.load_gather(zd2, [rows, cv]))
      out_v[pl.ds(g * 16, 16)] = acc
      return c2

    lax.fori_loop(0, CHUNK // 16, group, 0)
    pltpu.sync_copy(out_v, out_hbm.at[pl.ds((ib + v) * CHUNK, CHUNK)])
    return carry

  lax.fori_loop(0, L_CPW, visit, 0)
  g_wait(L_CPW)


def _layer1_body(aggp, xr, w1l, w1r, b1, w2l, w2r, b2, hw, hr, ic):
  a_lo = aggp[0]
  a_hi = aggp[1]
  inv = 1.0 / jnp.maximum(a_hi[:, CNT_COL], 1.0)
  m_lo = a_lo * inv[:, None]
  m_hi = a_hi[:, :CNT_COL] * inv[:, None]
  dn = (((1,), (1,)), ((), ()))
  f32 = jnp.float32
  h = (lax.dot_general(m_lo, w1l[:, :HALF_W], dn, preferred_element_type=f32)
       + lax.dot_general(m_hi, w1l[:, HALF_W:], dn, preferred_element_type=f32)
       + lax.dot_general(xr[...], w1r[...], dn, preferred_element_type=f32)
       + b1[...])
  h = jnp.maximum(h, 0.0)
  hw[...] = lax.dot_general(h, w2l[...], dn, preferred_element_type=f32)
  hr[...] = (lax.dot_general(h, w2r[...], dn, preferred_element_type=f32)
             + b2[...])
  ic[...] = inv[:, None]


def _layer1(aggp, x, W1_l, W1_r, b1, W2_l, W2_r, b2):
  BR = 1000
  return pl.pallas_call(
      lambda *refs: _layer1_body(refs[0][...], refs[1], refs[2][...],
                                 *refs[3:]),
      grid=(N_NODES // BR,),
      in_specs=[
          pl.BlockSpec((NC, BR, HALF_W), lambda i: (0, i, 0)),
          pl.BlockSpec((BR, IN_DIM), lambda i: (i, 0)),
          pl.BlockSpec((HID_DIM, IN_DIM), lambda i: (0, 0)),
          pl.BlockSpec((HID_DIM, IN_DIM), lambda i: (0, 0)),
          pl.BlockSpec((1, HID_DIM), lambda i: (0, 0)),
          pl.BlockSpec((OUT_DIM, HID_DIM), lambda i: (0, 0)),
          pl.BlockSpec((OUT_DIM, HID_DIM), lambda i: (0, 0)),
          pl.BlockSpec((1, OUT_DIM), lambda i: (0, 0)),
      ],
      out_specs=[
          pl.BlockSpec((BR, OUT_DIM), lambda i: (i, 0)),
          pl.BlockSpec((BR, OUT_DIM), lambda i: (i, 0)),
          pl.BlockSpec((BR, 1), lambda i: (i, 0)),
      ],
      out_shape=[
          jax.ShapeDtypeStruct((N_NODES, OUT_DIM), jnp.float32),
          jax.ShapeDtypeStruct((N_NODES, OUT_DIM), jnp.float32),
          jax.ShapeDtypeStruct((N_NODES, 1), jnp.float32),
      ],
  )(aggp, x, W1_l, W1_r, b1, W2_l, W2_r, b2)


def _assemble_body(aggp2, ic, hr, z):
  s = aggp2[0] + aggp2[1]
  z[...] = s * ic[...] + hr[...]


def _assemble_z(aggp2, ic, hr):
  BR = 1000
  return pl.pallas_call(
      lambda *refs: _assemble_body(refs[0][...], *refs[1:]),
      grid=(N_NODES // BR,),
      in_specs=[
          pl.BlockSpec((NC, BR, OUT_DIM), lambda i: (0, i, 0)),
          pl.BlockSpec((BR, 1), lambda i: (i, 0)),
          pl.BlockSpec((BR, OUT_DIM), lambda i: (i, 0)),
      ],
      out_specs=pl.BlockSpec((BR, OUT_DIM), lambda i: (i, 0)),
      out_shape=jax.ShapeDtypeStruct((N_NODES, OUT_DIM), jnp.float32),
  )(aggp2, ic, hr)


def kernel(x, edge_index, edge_label_index, W1_l, W1_r, b1, W2_l, W2_r, b2):
  i32 = jnp.int32
  f32 = jnp.float32
  src = edge_index[0].astype(i32)
  dst = edge_index[1].astype(i32)
  ls = edge_label_index[0].astype(i32)
  ld = edge_label_index[1].astype(i32)

  e1p = E1_PAD - N_EDGES
  src1 = jnp.concatenate([src, jnp.zeros((e1p,), i32)]).reshape(
      NS, E1_CPW, CHUNK)
  dst1 = jnp.concatenate([dst, jnp.full((e1p,), N_PAD - 1, i32)]).reshape(
      NS, E1_CPW, CHUNK)
  e2p = E2_PAD - N_EDGES
  src2 = jnp.concatenate([src, jnp.zeros((e2p,), i32)]).reshape(
      NW, E2_CPW, CHUNK)
  dst2 = jnp.concatenate([dst, jnp.full((e2p,), N_PAD - 1, i32)]).reshape(
      NW, E2_CPW, CHUNK)
  lp = L_PAD - N_LABEL
  ls2 = jnp.concatenate([ls, jnp.zeros((lp,), i32)]).reshape(NW, L_CPW, CHUNK)
  ld2 = jnp.concatenate([ld, jnp.zeros((lp,), i32)]).reshape(NW, L_CPW, CHUNK)

  xab = jnp.stack([
      x[:, :HALF_W],
      jnp.concatenate([x[:, HALF_W:], jnp.ones((N_NODES, 1), f32),
                       jnp.zeros((N_NODES, HALF_W - CNT_COL - 1), f32)],
                      axis=1),
  ])

  aggp1 = _segsum_l1(xab, src1, dst1)
  hw, hr, ic = _layer1(aggp1, x, W1_l, W1_r, b1.reshape(1, HID_DIM),
                       W2_l, W2_r, b2.reshape(1, OUT_DIM))
  aggp2 = _segsum_l2(hw, src2, dst2)
  z = _assemble_z(aggp2, ic, hr)
  out = _decode(z, ls2, ld2)
  return out[:N_LABEL]


# speedup vs baseline: 5.5003x; 1.1002x over previous
import functools

import jax
import jax.numpy as jnp
from jax import lax
from jax.experimental import pallas as pl
from jax.experimental.pallas import tpu as pltpu
from jax.experimental.pallas import tpu_sc as plsc

N_NODES = 10000
IN_DIM = 128
HID_DIM = 128
OUT_DIM = 64
N_EDGES = 320000
N_LABEL = 100000

NC, NS = 2, 16
NW = NC * NS
CHUNK = 128

N_PAD = 10240
ROWS_PER_SUB = N_PAD // NS

HALF_W = 72
CNT_COL = IN_DIM - HALF_W

E1_CPW = 157
E1_PAD = NS * E1_CPW * CHUNK

E2_CPW = 80
E2_PAD = NW * E2_CPW * CHUNK

L_CPW = 25
L_PAD = NW * L_CPW * CHUNK

_MESH = plsc.VectorSubcoreMesh(core_axis_name="c", subcore_axis_name="s")
_SC_PARAMS = pltpu.CompilerParams(use_tc_tiling_on_sc=False,
                                  needs_layout_passes=False)


def _zero_rows(buf, width):
  z16 = jnp.zeros((16,), jnp.float32)
  ncol = (width + 15) // 16

  def zrow(r, carry):
    for c in range(ncol):
      buf[r, pl.ds(min(c * 16, width - 16), 16)] = z16
    return carry

  lax.fori_loop(0, CHUNK, zrow, 0)


def _make_segsum(width, cpw, split_cols):
  @functools.partial(
      pl.kernel,
      out_type=jax.ShapeDtypeStruct((NC, N_PAD, width), jnp.float32),
      mesh=_MESH,
      compiler_params=_SC_PARAMS,
      scratch_types=[
          pltpu.VMEM((cpw, CHUNK), jnp.int32),
          pltpu.VMEM((cpw, CHUNK), jnp.int32),
          pltpu.VMEM((4 * CHUNK, width), jnp.float32),
          pltpu.VMEM_SHARED((N_PAD, width), jnp.float32),
          pltpu.SemaphoreType.DMA,
          pltpu.SemaphoreType.DMA,
      ],
  )
  def segsum(tab_hbm, src_hbm, dst_hbm, out_hbm,
             src_v, dst_v, rows2, acc_sh, gsem, ssem):
    cid = lax.axis_index("c")
    sid = lax.axis_index("s")
    r0 = sid * ROWS_PER_SUB
    tab = tab_hbm.at[cid] if split_cols else tab_hbm
    isel = sid if split_cols else cid * NS + sid

    _zero_rows(rows2, width)
    for t in range(ROWS_PER_SUB // CHUNK):
      pltpu.sync_copy(rows2.at[pl.ds(0, CHUNK)],
                      acc_sh.at[pl.ds(r0 + t * CHUNK, CHUNK)])
    plsc.subcore_barrier()
    pltpu.sync_copy(src_hbm.at[isel], src_v)
    pltpu.sync_copy(dst_hbm.at[isel], dst_v)

    def buf(v):
      return rows2.at[pl.ds((v % 4) * CHUNK, CHUNK)]

    def g_start(j, v):
      pltpu.async_copy(tab.at[src_v.at[j]], buf(v), gsem)

    def g_wait(v):
      pltpu.make_async_copy(tab.at[src_v.at[0]], buf(v), gsem).wait()

    def s_start(v):
      pltpu.async_copy(buf(v), acc_sh.at[dst_v.at[v]], ssem, add=True)

    def s_wait():
      pltpu.make_async_copy(buf(0), acc_sh.at[dst_v.at[0]], ssem).wait()

    g_start(0, 0)
    g_start(1, 1)
    g_wait(0)
    s_start(0)
    g_start(2, 2)
    g_wait(1)
    s_start(1)
    g_start(3, 3)

    def body(v, carry):
      g_wait(v)
      s_wait()
      s_start(v)
      g_start(jnp.minimum(v + 2, cpw - 1), v + 2)
      return carry

    lax.fori_loop(2, cpw, body, 0)
    g_wait(cpw)
    g_wait(cpw + 1)
    s_wait()
    s_wait()
    plsc.subcore_barrier()
    pltpu.sync_copy(acc_sh.at[pl.ds(r0, ROWS_PER_SUB)],
                    out_hbm.at[cid, pl.ds(r0, ROWS_PER_SUB)])

  return segsum


_segsum_l1 = _make_segsum(HALF_W, E1_CPW, split_cols=True)
_segsum_l2 = _make_segsum(OUT_DIM, E2_CPW, split_cols=False)


@functools.partial(
    pl.kernel,
    out_type=jax.ShapeDtypeStruct((L_PAD,), jnp.float32),
    mesh=_MESH,
    compiler_params=_SC_PARAMS,
    scratch_types=[
        pltpu.VMEM((L_CPW, CHUNK), jnp.int32),
        pltpu.VMEM((L_CPW, CHUNK), jnp.int32),
        pltpu.VMEM((3 * CHUNK, OUT_DIM), jnp.float32),
        pltpu.VMEM((3 * CHUNK, OUT_DIM), jnp.float32),
        pltpu.VMEM((CHUNK,), jnp.float32),
        pltpu.SemaphoreType.DMA,
        pltpu.SemaphoreType.DMA,
    ],
)
def _decode(z_hbm, ls_hbm, ld_hbm, out_hbm, ls_v, ld_v, zs2, zd2, out_v,
            ssm, dsm):
  cid = lax.axis_index("c")
  sid = lax.axis_index("s")
  wid = cid * NS + sid
  ib = wid * L_CPW
  pltpu.sync_copy(ls_hbm.at[wid], ls_v)
  pltpu.sync_copy(ld_hbm.at[wid], ld_v)

  def g_start(j, v):
    boff = (v % 3) * CHUNK
    pltpu.async_copy(z_hbm.at[ls_v.at[j]], zs2.at[pl.ds(boff, CHUNK)], ssm)
    pltpu.async_copy(z_hbm.at[ld_v.at[j]], zd2.at[pl.ds(boff, CHUNK)], dsm)

  def g_wait(v):
    boff = (v % 3) * CHUNK
    pltpu.make_async_copy(z_hbm.at[ls_v.at[0]], zs2.at[pl.ds(boff, CHUNK)],
                          ssm).wait()
    pltpu.make_async_copy(z_hbm.at[ld_v.at[0]], zd2.at[pl.ds(boff, CHUNK)],
                          dsm).wait()

  g_start(0, 0)
  g_start(1, 1)

  def visit(v, carry):
    g_wait(v)
    g_start(jnp.minimum(v + 2, L_CPW - 1), v + 2)
    boff = (v % 3) * CHUNK

    def group(g, c2):
      rows = boff + g * 16 + lax.iota(jnp.int32, 16)
      acc = jnp.zeros((16,), jnp.float32)
      for col in range(OUT_DIM):
        cv = jnp.full((16,), col, jnp.int32)
        acc = acc + (plsc.load_gather(zs2, [rows, cv]) *
                     plsc.load_gather(zd2, [rows, cv]))
      out_v[pl.ds(g * 16, 16)] = acc
      return c2

    lax.fori_loop(0, CHUNK // 16, group, 0)
    pltpu.sync_copy(out_v, out_hbm.at[pl.ds((ib + v) * CHUNK, CHUNK)])
    return carry

  lax.fori_loop(0, L_CPW, visit, 0)
  g_wait(L_CPW)
  g_wait(L_CPW + 1)


def _layer1_body(aggp, xr, w1l, w1r, b1, w2l, w2r, b2, hw, hr, ic):
  a_lo = aggp[0]
  a_hi = aggp[1]
  inv = 1.0 / jnp.maximum(a_hi[:, CNT_COL], 1.0)
  m_lo = a_lo * inv[:, None]
  m_hi = a_hi[:, :CNT_COL] * inv[:, None]
  dn = (((1,), (1,)), ((), ()))
  f32 = jnp.float32
  h = (lax.dot_general(m_lo, w1l[:, :HALF_W], dn, preferred_element_type=f32)
       + lax.dot_general(m_hi, w1l[:, HALF_W:], dn, preferred_element_type=f32)
       + lax.dot_general(xr[...], w1r[...], dn, preferred_element_type=f32)
       + b1[...])
  h = jnp.maximum(h, 0.0)
  hw[...] = lax.dot_general(h, w2l[...], dn, preferred_element_type=f32)
  hr[...] = (lax.dot_general(h, w2r[...], dn, preferred_element_type=f32)
             + b2[...])
  ic[...] = inv[:, None]


def _layer1(aggp, x, W1_l, W1_r, b1, W2_l, W2_r, b2):
  BR = 1000
  return pl.pallas_call(
      lambda *refs: _layer1_body(refs[0][...], refs[1], refs[2][...],
                                 *refs[3:]),
      grid=(N_NODES // BR,),
      in_specs=[
          pl.BlockSpec((NC, BR, HALF_W), lambda i: (0, i, 0)),
          pl.BlockSpec((BR, IN_DIM), lambda i: (i, 0)),
          pl.BlockSpec((HID_DIM, IN_DIM), lambda i: (0, 0)),
          pl.BlockSpec((HID_DIM, IN_DIM), lambda i: (0, 0)),
          pl.BlockSpec((1, HID_DIM), lambda i: (0, 0)),
          pl.BlockSpec((OUT_DIM, HID_DIM), lambda i: (0, 0)),
          pl.BlockSpec((OUT_DIM, HID_DIM), lambda i: (0, 0)),
          pl.BlockSpec((1, OUT_DIM), lambda i: (0, 0)),
      ],
      out_specs=[
          pl.BlockSpec((BR, OUT_DIM), lambda i: (i, 0)),
          pl.BlockSpec((BR, OUT_DIM), lambda i: (i, 0)),
          pl.BlockSpec((BR, 1), lambda i: (i, 0)),
      ],
      out_shape=[
          jax.ShapeDtypeStruct((N_NODES, OUT_DIM), jnp.float32),
          jax.ShapeDtypeStruct((N_NODES, OUT_DIM), jnp.float32),
          jax.ShapeDtypeStruct((N_NODES, 1), jnp.float32),
      ],
  )(aggp, x, W1_l, W1_r, b1, W2_l, W2_r, b2)


def _assemble_body(aggp2, ic, hr, z):
  s = aggp2[0] + aggp2[1]
  z[...] = s * ic[...] + hr[...]


def _assemble_z(aggp2, ic, hr):
  BR = 1000
  return pl.pallas_call(
      lambda *refs: _assemble_body(refs[0][...], *refs[1:]),
      grid=(N_NODES // BR,),
      in_specs=[
          pl.BlockSpec((NC, BR, OUT_DIM), lambda i: (0, i, 0)),
          pl.BlockSpec((BR, 1), lambda i: (i, 0)),
          pl.BlockSpec((BR, OUT_DIM), lambda i: (i, 0)),
      ],
      out_specs=pl.BlockSpec((BR, OUT_DIM), lambda i: (i, 0)),
      out_shape=jax.ShapeDtypeStruct((N_NODES, OUT_DIM), jnp.float32),
  )(aggp2, ic, hr)


def kernel(x, edge_index, edge_label_index, W1_l, W1_r, b1, W2_l, W2_r, b2):
  i32 = jnp.int32
  f32 = jnp.float32
  src = edge_index[0].astype(i32)
  dst = edge_index[1].astype(i32)
  ls = edge_label_index[0].astype(i32)
  ld = edge_label_index[1].astype(i32)

  e1p = E1_PAD - N_EDGES
  src1 = jnp.concatenate([src, jnp.zeros((e1p,), i32)]).reshape(
      NS, E1_CPW, CHUNK)
  dst1 = jnp.concatenate([dst, jnp.full((e1p,), N_PAD - 1, i32)]).reshape(
      NS, E1_CPW, CHUNK)
  e2p = E2_PAD - N_EDGES
  src2 = jnp.concatenate([src, jnp.zeros((e2p,), i32)]).reshape(
      NW, E2_CPW, CHUNK)
  dst2 = jnp.concatenate([dst, jnp.full((e2p,), N_PAD - 1, i32)]).reshape(
      NW, E2_CPW, CHUNK)
  lp = L_PAD - N_LABEL
  ls2 = jnp.concatenate([ls, jnp.zeros((lp,), i32)]).reshape(NW, L_CPW, CHUNK)
  ld2 = jnp.concatenate([ld, jnp.zeros((lp,), i32)]).reshape(NW, L_CPW, CHUNK)

  xab = jnp.stack([
      x[:, :HALF_W],
      jnp.concatenate([x[:, HALF_W:], jnp.ones((N_NODES, 1), f32),
                       jnp.zeros((N_NODES, HALF_W - CNT_COL - 1), f32)],
                      axis=1),
  ])

  aggp1 = _segsum_l1(xab, src1, dst1)
  hw, hr, ic = _layer1(aggp1, x, W1_l, W1_r, b1.reshape(1, HID_DIM),
                       W2_l, W2_r, b2.reshape(1, OUT_DIM))
  aggp2 = _segsum_l2(hw, src2, dst2)
  z = _assemble_z(aggp2, ic, hr)
  out = _decode(z, ls2, ld2)
  return out[:N_LABEL]


# speedup vs baseline: 5.6924x; 1.0349x over previous
import functools

import jax
import jax.numpy as jnp
from jax import lax
from jax.experimental import pallas as pl
from jax.experimental.pallas import tpu as pltpu
from jax.experimental.pallas import tpu_sc as plsc

N_NODES = 10000
IN_DIM = 128
HID_DIM = 128
OUT_DIM = 64
N_EDGES = 320000
N_LABEL = 100000

NC, NS = 2, 16
NW = NC * NS
CHUNK = 128

N_PAD = 10240
ROWS_PER_SUB = N_PAD // NS

HALF_W = 72
CNT_COL = IN_DIM - HALF_W

E1_CPW = 157
E1_PAD = NS * E1_CPW * CHUNK

E2_CPW = 80
E2_PAD = NW * E2_CPW * CHUNK

L_CPW = 25
L_PAD = NW * L_CPW * CHUNK

_MESH = plsc.VectorSubcoreMesh(core_axis_name="c", subcore_axis_name="s")
_SC_PARAMS = pltpu.CompilerParams(use_tc_tiling_on_sc=False,
                                  needs_layout_passes=False)


def _zero_rows(buf, width):
  z16 = jnp.zeros((16,), jnp.float32)
  ncol = (width + 15) // 16

  def zrow(r, carry):
    for c in range(ncol):
      buf[r, pl.ds(min(c * 16, width - 16), 16)] = z16
    return carry

  lax.fori_loop(0, CHUNK, zrow, 0)


def _make_segsum(width, cpw, split_cols):
  @functools.partial(
      pl.kernel,
      out_type=jax.ShapeDtypeStruct((NC, N_PAD, width), jnp.float32),
      mesh=_MESH,
      compiler_params=_SC_PARAMS,
      scratch_types=[
          pltpu.VMEM((cpw, CHUNK), jnp.int32),
          pltpu.VMEM((cpw, CHUNK), jnp.int32),
          pltpu.VMEM((4 * CHUNK, width), jnp.float32),
          pltpu.VMEM_SHARED((N_PAD, width), jnp.float32),
          pltpu.SemaphoreType.DMA,
          pltpu.SemaphoreType.DMA,
      ],
  )
  def segsum(tab_hbm, src_hbm, dst_hbm, out_hbm,
             src_v, dst_v, rows2, acc_sh, gsem, ssem):
    cid = lax.axis_index("c")
    sid = lax.axis_index("s")
    r0 = sid * ROWS_PER_SUB
    tab = tab_hbm.at[cid] if split_cols else tab_hbm
    isel = sid if split_cols else cid * NS + sid

    _zero_rows(rows2, width)
    for t in range(ROWS_PER_SUB // CHUNK):
      pltpu.sync_copy(rows2.at[pl.ds(0, CHUNK)],
                      acc_sh.at[pl.ds(r0 + t * CHUNK, CHUNK)])
    plsc.subcore_barrier()
    pltpu.sync_copy(src_hbm.at[isel], src_v)
    pltpu.sync_copy(dst_hbm.at[isel], dst_v)

    def buf(v):
      return rows2.at[pl.ds((v % 4) * CHUNK, CHUNK)]

    def g_start(j, v):
      pltpu.async_copy(tab.at[src_v.at[j]], buf(v), gsem)

    def g_wait(v):
      pltpu.make_async_copy(tab.at[src_v.at[0]], buf(v), gsem).wait()

    def s_start(v):
      pltpu.async_copy(buf(v), acc_sh.at[dst_v.at[v]], ssem, add=True)

    def s_wait():
      pltpu.make_async_copy(buf(0), acc_sh.at[dst_v.at[0]], ssem).wait()

    def body(v, carry):
      u = v - 2

      @pl.when(v >= 2)
      def _process():
        g_wait(u)

        @pl.when(u >= 2)
        def _reclaim():
          s_wait()

        s_start(u)

      @pl.when(v < cpw)
      def _fetch():
        g_start(v, v)

      return carry

    lax.fori_loop(0, cpw + 2, body, 0)
    for _ in range(2):
      s_wait()
    plsc.subcore_barrier()
    pltpu.sync_copy(acc_sh.at[pl.ds(r0, ROWS_PER_SUB)],
                    out_hbm.at[cid, pl.ds(r0, ROWS_PER_SUB)])

  return segsum


_segsum_l1 = _make_segsum(HALF_W, E1_CPW, split_cols=True)
_segsum_l2 = _make_segsum(OUT_DIM, E2_CPW, split_cols=False)


@functools.partial(
    pl.kernel,
    out_type=jax.ShapeDtypeStruct((L_PAD,), jnp.float32),
    mesh=_MESH,
    compiler_params=_SC_PARAMS,
    scratch_types=[
        pltpu.VMEM((L_CPW, CHUNK), jnp.int32),
        pltpu.VMEM((L_CPW, CHUNK), jnp.int32),
        pltpu.VMEM((4 * CHUNK, OUT_DIM), jnp.float32),
        pltpu.VMEM((4 * CHUNK, OUT_DIM), jnp.float32),
        pltpu.VMEM((L_CPW * CHUNK,), jnp.float32),
        pltpu.SemaphoreType.DMA,
        pltpu.SemaphoreType.DMA,
    ],
)
def _decode(z_hbm, ls_hbm, ld_hbm, out_hbm, ls_v, ld_v, zs2, zd2, out_v,
            ssm, dsm):
  cid = lax.axis_index("c")
  sid = lax.axis_index("s")
  wid = cid * NS + sid
  ib = wid * L_CPW
  pltpu.sync_copy(ls_hbm.at[wid], ls_v)
  pltpu.sync_copy(ld_hbm.at[wid], ld_v)

  def g_start(j, v):
    boff = (v % 4) * CHUNK
    pltpu.async_copy(z_hbm.at[ls_v.at[j]], zs2.at[pl.ds(boff, CHUNK)], ssm)
    pltpu.async_copy(z_hbm.at[ld_v.at[j]], zd2.at[pl.ds(boff, CHUNK)], dsm)

  def g_wait(v):
    boff = (v % 4) * CHUNK
    pltpu.make_async_copy(z_hbm.at[ls_v.at[0]], zs2.at[pl.ds(boff, CHUNK)],
                          ssm).wait()
    pltpu.make_async_copy(z_hbm.at[ld_v.at[0]], zd2.at[pl.ds(boff, CHUNK)],
                          dsm).wait()

  def visit(v, carry):
    u = v - 3

    @pl.when(v >= 3)
    def _process():
      g_wait(u)
      boff = (u % 4) * CHUNK

      def group(g, c2):
        rows = boff + g * 16 + lax.iota(jnp.int32, 16)
        acc = jnp.zeros((16,), jnp.float32)
        for col in range(OUT_DIM):
          cv = jnp.full((16,), col, jnp.int32)
          acc = acc + (plsc.load_gather(zs2, [rows, cv]) *
                       plsc.load_gather(zd2, [rows, cv]))
        out_v[pl.ds(u * CHUNK + g * 16, 16)] = acc
        return c2

      lax.fori_loop(0, CHUNK // 16, group, 0)

    @pl.when(v < L_CPW)
    def _fetch():
      g_start(v, v)

    return carry

  lax.fori_loop(0, L_CPW + 3, visit, 0)
  pltpu.sync_copy(out_v, out_hbm.at[pl.ds(ib * CHUNK, L_CPW * CHUNK)])


def _layer1_body(aggp, xr, w1l, w1r, b1, w2l, w2r, b2, hw, hr, ic):
  a_lo = aggp[0]
  a_hi = aggp[1]
  inv = 1.0 / jnp.maximum(a_hi[:, CNT_COL], 1.0)
  m_lo = a_lo * inv[:, None]
  m_hi = a_hi[:, :CNT_COL] * inv[:, None]
  dn = (((1,), (1,)), ((), ()))
  f32 = jnp.float32
  h = (lax.dot_general(m_lo, w1l[:, :HALF_W], dn, preferred_element_type=f32)
       + lax.dot_general(m_hi, w1l[:, HALF_W:], dn, preferred_element_type=f32)
       + lax.dot_general(xr[...], w1r[...], dn, preferred_element_type=f32)
       + b1[...])
  h = jnp.maximum(h, 0.0)
  hw[...] = lax.dot_general(h, w2l[...], dn, preferred_element_type=f32)
  hr[...] = (lax.dot_general(h, w2r[...], dn, preferred_element_type=f32)
             + b2[...])
  ic[...] = inv[:, None]


def _layer1(aggp, x, W1_l, W1_r, b1, W2_l, W2_r, b2):
  BR = 1000
  return pl.pallas_call(
      lambda *refs: _layer1_body(refs[0][...], refs[1], refs[2][...],
                                 *refs[3:]),
      grid=(N_NODES // BR,),
      in_specs=[
          pl.BlockSpec((NC, BR, HALF_W), lambda i: (0, i, 0)),
          pl.BlockSpec((BR, IN_DIM), lambda i: (i, 0)),
          pl.BlockSpec((HID_DIM, IN_DIM), lambda i: (0, 0)),
          pl.BlockSpec((HID_DIM, IN_DIM), lambda i: (0, 0)),
          pl.BlockSpec((1, HID_DIM), lambda i: (0, 0)),
          pl.BlockSpec((OUT_DIM, HID_DIM), lambda i: (0, 0)),
          pl.BlockSpec((OUT_DIM, HID_DIM), lambda i: (0, 0)),
          pl.BlockSpec((1, OUT_DIM), lambda i: (0, 0)),
      ],
      out_specs=[
          pl.BlockSpec((BR, OUT_DIM), lambda i: (i, 0)),
          pl.BlockSpec((BR, OUT_DIM), lambda i: (i, 0)),
          pl.BlockSpec((BR, 1), lambda i: (i, 0)),
      ],
      out_shape=[
          jax.ShapeDtypeStruct((N_NODES, OUT_DIM), jnp.float32),
          jax.ShapeDtypeStruct((N_NODES, OUT_DIM), jnp.float32),
          jax.ShapeDtypeStruct((N_NODES, 1), jnp.float32),
      ],
  )(aggp, x, W1_l, W1_r, b1, W2_l, W2_r, b2)


def _assemble_body(aggp2, ic, hr, z):
  s = aggp2[0] + aggp2[1]
  z[...] = s * ic[...] + hr[...]


def _assemble_z(aggp2, ic, hr):
  BR = 1000
  return pl.pallas_call(
      lambda *refs: _assemble_body(refs[0][...], *refs[1:]),
      grid=(N_NODES // BR,),
      in_specs=[
          pl.BlockSpec((NC, BR, OUT_DIM), lambda i: (0, i, 0)),
          pl.BlockSpec((BR, 1), lambda i: (i, 0)),
          pl.BlockSpec((BR, OUT_DIM), lambda i: (i, 0)),
      ],
      out_specs=pl.BlockSpec((BR, OUT_DIM), lambda i: (i, 0)),
      out_shape=jax.ShapeDtypeStruct((N_NODES, OUT_DIM), jnp.float32),
  )(aggp2, ic, hr)


def kernel(x, edge_index, edge_label_index, W1_l, W1_r, b1, W2_l, W2_r, b2):
  i32 = jnp.int32
  f32 = jnp.float32
  src = edge_index[0].astype(i32)
  dst = edge_index[1].astype(i32)
  ls = edge_label_index[0].astype(i32)
  ld = edge_label_index[1].astype(i32)

  e1p = E1_PAD - N_EDGES
  src1 = jnp.concatenate([src, jnp.zeros((e1p,), i32)]).reshape(
      NS, E1_CPW, CHUNK)
  dst1 = jnp.concatenate([dst, jnp.full((e1p,), N_PAD - 1, i32)]).reshape(
      NS, E1_CPW, CHUNK)
  e2p = E2_PAD - N_EDGES
  src2 = jnp.concatenate([src, jnp.zeros((e2p,), i32)]).reshape(
      NW, E2_CPW, CHUNK)
  dst2 = jnp.concatenate([dst, jnp.full((e2p,), N_PAD - 1, i32)]).reshape(
      NW, E2_CPW, CHUNK)
  lp = L_PAD - N_LABEL
  ls2 = jnp.concatenate([ls, jnp.zeros((lp,), i32)]).reshape(NW, L_CPW, CHUNK)
  ld2 = jnp.concatenate([ld, jnp.zeros((lp,), i32)]).reshape(NW, L_CPW, CHUNK)

  xab = jnp.stack([
      x[:, :HALF_W],
      jnp.concatenate([x[:, HALF_W:], jnp.ones((N_NODES, 1), f32),
                       jnp.zeros((N_NODES, HALF_W - CNT_COL - 1), f32)],
                      axis=1),
  ])

  aggp1 = _segsum_l1(xab, src1, dst1)
  hw, hr, ic = _layer1(aggp1, x, W1_l, W1_r, b1.reshape(1, HID_DIM),
                       W2_l, W2_r, b2.reshape(1, OUT_DIM))
  aggp2 = _segsum_l2(hw, src2, dst2)
  z = _assemble_z(aggp2, ic, hr)
  out = _decode(z, ls2, ld2)
  return out[:N_LABEL]


# speedup vs baseline: 7.4241x; 1.3042x over previous
import functools

import jax
import jax.numpy as jnp
from jax import lax
from jax.experimental import pallas as pl
from jax.experimental.pallas import tpu as pltpu
from jax.experimental.pallas import tpu_sc as plsc

N_NODES = 10000
IN_DIM = 128
HID_DIM = 128
OUT_DIM = 64
N_EDGES = 320000
N_LABEL = 100000

NC, NS = 2, 16
NW = NC * NS
CHUNK = 128

N_PAD = 10240
ROWS_PER_SUB = N_PAD // NS

HALF_W = 72
CNT_COL = IN_DIM - HALF_W

E1_CPW = 157
E1_PAD = NS * E1_CPW * CHUNK

L_CPW = 25
L_PAD = NW * L_CPW * CHUNK

_MESH = plsc.VectorSubcoreMesh(core_axis_name="c", subcore_axis_name="s")
_SC_PARAMS = pltpu.CompilerParams(use_tc_tiling_on_sc=False,
                                  needs_layout_passes=False)


def _zero_rows(buf, width):
  z16 = jnp.zeros((16,), jnp.float32)
  ncol = (width + 15) // 16

  def zrow(r, carry):
    for c in range(ncol):
      buf[r, pl.ds(min(c * 16, width - 16), 16)] = z16
    return carry

  lax.fori_loop(0, CHUNK, zrow, 0)


def _make_segsum(width, cpw, split_cols, depth_g=2, depth_s=2):
  @functools.partial(
      pl.kernel,
      out_type=jax.ShapeDtypeStruct((NC, N_PAD, width), jnp.float32),
      mesh=_MESH,
      compiler_params=_SC_PARAMS,
      scratch_types=[
          pltpu.VMEM((cpw, CHUNK), jnp.int32),
          pltpu.VMEM((cpw, CHUNK), jnp.int32),
          pltpu.VMEM(((depth_g + depth_s) * CHUNK, width), jnp.float32),
          pltpu.VMEM_SHARED((N_PAD, width), jnp.float32),
          pltpu.SemaphoreType.DMA,
          pltpu.SemaphoreType.DMA,
      ],
  )
  def segsum(tab_hbm, src_hbm, dst_hbm, out_hbm,
             src_v, dst_v, rows2, acc_sh, gsem, ssem):
    cid = lax.axis_index("c")
    sid = lax.axis_index("s")
    r0 = sid * ROWS_PER_SUB
    tab = tab_hbm.at[cid] if split_cols else tab_hbm
    isel = sid if split_cols else cid * NS + sid

    _zero_rows(rows2, width)
    for t in range(ROWS_PER_SUB // CHUNK):
      pltpu.sync_copy(rows2.at[pl.ds(0, CHUNK)],
                      acc_sh.at[pl.ds(r0 + t * CHUNK, CHUNK)])
    plsc.subcore_barrier()
    pltpu.sync_copy(src_hbm.at[isel], src_v)
    pltpu.sync_copy(dst_hbm.at[isel], dst_v)

    nbuf = depth_g + depth_s

    def buf(v):
      return rows2.at[pl.ds((v % nbuf) * CHUNK, CHUNK)]

    def g_start(j, v):
      pltpu.async_copy(tab.at[src_v.at[j]], buf(v), gsem)

    def g_wait(v):
      pltpu.make_async_copy(tab.at[src_v.at[0]], buf(v), gsem).wait()

    def s_start(v):
      pltpu.async_copy(buf(v), acc_sh.at[dst_v.at[v]], ssem, add=True)

    def s_wait():
      pltpu.make_async_copy(buf(0), acc_sh.at[dst_v.at[0]], ssem).wait()

    def body(v, carry):
      u = v - depth_g

      @pl.when(v >= depth_g)
      def _process():
        g_wait(u)

        @pl.when(u >= depth_s)
        def _reclaim():
          s_wait()

        s_start(u)

      @pl.when(v < cpw)
      def _fetch():
        g_start(v, v)

      return carry

    lax.fori_loop(0, cpw + depth_g, body, 0)
    for _ in range(depth_s):
      s_wait()
    plsc.subcore_barrier()
    pltpu.sync_copy(acc_sh.at[pl.ds(r0, ROWS_PER_SUB)],
                    out_hbm.at[cid, pl.ds(r0, ROWS_PER_SUB)])

  return segsum


_segsum_l1 = _make_segsum(HALF_W, E1_CPW, split_cols=True)
_segsum_l2 = _make_segsum(OUT_DIM // 2, E1_CPW, split_cols=True,
                          depth_g=3, depth_s=2)


@functools.partial(
    pl.kernel,
    out_type=jax.ShapeDtypeStruct((L_PAD,), jnp.float32),
    mesh=_MESH,
    compiler_params=_SC_PARAMS,
    scratch_types=[
        pltpu.VMEM((L_CPW, CHUNK), jnp.int32),
        pltpu.VMEM((L_CPW, CHUNK), jnp.int32),
        pltpu.VMEM((4 * CHUNK, OUT_DIM), jnp.float32),
        pltpu.VMEM((4 * CHUNK, OUT_DIM), jnp.float32),
        pltpu.VMEM((L_CPW * CHUNK,), jnp.float32),
        pltpu.SemaphoreType.DMA,
        pltpu.SemaphoreType.DMA,
    ],
)
def _decode(z_hbm, ls_hbm, ld_hbm, out_hbm, ls_v, ld_v, zs2, zd2, out_v,
            ssm, dsm):
  cid = lax.axis_index("c")
  sid = lax.axis_index("s")
  wid = cid * NS + sid
  ib = wid * L_CPW
  pltpu.sync_copy(ls_hbm.at[wid], ls_v)
  pltpu.sync_copy(ld_hbm.at[wid], ld_v)

  def g_start(j, v):
    boff = (v % 4) * CHUNK
    pltpu.async_copy(z_hbm.at[ls_v.at[j]], zs2.at[pl.ds(boff, CHUNK)], ssm)
    pltpu.async_copy(z_hbm.at[ld_v.at[j]], zd2.at[pl.ds(boff, CHUNK)], dsm)

  def g_wait(v):
    boff = (v % 4) * CHUNK
    pltpu.make_async_copy(z_hbm.at[ls_v.at[0]], zs2.at[pl.ds(boff, CHUNK)],
                          ssm).wait()
    pltpu.make_async_copy(z_hbm.at[ld_v.at[0]], zd2.at[pl.ds(boff, CHUNK)],
                          dsm).wait()

  def visit(v, carry):
    u = v - 3

    @pl.when(v >= 3)
    def _process():
      g_wait(u)
      boff = (u % 4) * CHUNK

      def group(g, c2):
        rows = boff + g * 16 + lax.iota(jnp.int32, 16)
        acc = jnp.zeros((16,), jnp.float32)
        for col in range(OUT_DIM):
          cv = jnp.full((16,), col, jnp.int32)
          acc = acc + (plsc.load_gather(zs2, [rows, cv]) *
                       plsc.load_gather(zd2, [rows, cv]))
        out_v[pl.ds(u * CHUNK + g * 16, 16)] = acc
        return c2

      lax.fori_loop(0, CHUNK // 16, group, 0)

    @pl.when(v < L_CPW)
    def _fetch():
      g_start(v, v)

    return carry

  lax.fori_loop(0, L_CPW + 3, visit, 0)
  pltpu.sync_copy(out_v, out_hbm.at[pl.ds(ib * CHUNK, L_CPW * CHUNK)])


def _layer1_body(aggp, xr, w1l, w1r, b1, w2l, w2r, b2, hw, hr, ic):
  a_lo = aggp[0]
  a_hi = aggp[1]
  inv = 1.0 / jnp.maximum(a_hi[:, CNT_COL], 1.0)
  m_lo = a_lo * inv[:, None]
  m_hi = a_hi[:, :CNT_COL] * inv[:, None]
  dn = (((1,), (1,)), ((), ()))
  f32 = jnp.float32
  h = (lax.dot_general(m_lo, w1l[:, :HALF_W], dn, preferred_element_type=f32)
       + lax.dot_general(m_hi, w1l[:, HALF_W:], dn, preferred_element_type=f32)
       + lax.dot_general(xr[...], w1r[...], dn, preferred_element_type=f32)
       + b1[...])
  h = jnp.maximum(h, 0.0)
  w2l_a = w2l[...]
  hw[0] = lax.dot_general(h, w2l_a[:OUT_DIM // 2], dn,
                          preferred_element_type=f32)
  hw[1] = lax.dot_general(h, w2l_a[OUT_DIM // 2:], dn,
                          preferred_element_type=f32)
  hr[...] = (lax.dot_general(h, w2r[...], dn, preferred_element_type=f32)
             + b2[...])
  ic[...] = inv[:, None]


def _layer1(aggp, x, W1_l, W1_r, b1, W2_l, W2_r, b2):
  BR = 1000
  return pl.pallas_call(
      lambda *refs: _layer1_body(refs[0][...], refs[1], refs[2][...],
                                 *refs[3:]),
      grid=(N_NODES // BR,),
      in_specs=[
          pl.BlockSpec((NC, BR, HALF_W), lambda i: (0, i, 0)),
          pl.BlockSpec((BR, IN_DIM), lambda i: (i, 0)),
          pl.BlockSpec((HID_DIM, IN_DIM), lambda i: (0, 0)),
          pl.BlockSpec((HID_DIM, IN_DIM), lambda i: (0, 0)),
          pl.BlockSpec((1, HID_DIM), lambda i: (0, 0)),
          pl.BlockSpec((OUT_DIM, HID_DIM), lambda i: (0, 0)),
          pl.BlockSpec((OUT_DIM, HID_DIM), lambda i: (0, 0)),
          pl.BlockSpec((1, OUT_DIM), lambda i: (0, 0)),
      ],
      out_specs=[
          pl.BlockSpec((NC, BR, OUT_DIM // 2), lambda i: (0, i, 0)),
          pl.BlockSpec((BR, OUT_DIM), lambda i: (i, 0)),
          pl.BlockSpec((BR, 1), lambda i: (i, 0)),
      ],
      out_shape=[
          jax.ShapeDtypeStruct((NC, N_NODES, OUT_DIM // 2), jnp.float32),
          jax.ShapeDtypeStruct((N_NODES, OUT_DIM), jnp.float32),
          jax.ShapeDtypeStruct((N_NODES, 1), jnp.float32),
      ],
  )(aggp, x, W1_l, W1_r, b1, W2_l, W2_r, b2)


def _assemble_body(aggp2, ic, hr, z):
  iv = ic[...]
  hra = hr[...]
  z[:, :OUT_DIM // 2] = aggp2[0] * iv + hra[:, :OUT_DIM // 2]
  z[:, OUT_DIM // 2:] = aggp2[1] * iv + hra[:, OUT_DIM // 2:]


def _assemble_z(aggp2, ic, hr):
  BR = 1000
  return pl.pallas_call(
      lambda *refs: _assemble_body(refs[0][...], *refs[1:]),
      grid=(N_NODES // BR,),
      in_specs=[
          pl.BlockSpec((NC, BR, OUT_DIM // 2), lambda i: (0, i, 0)),
          pl.BlockSpec((BR, 1), lambda i: (i, 0)),
          pl.BlockSpec((BR, OUT_DIM), lambda i: (i, 0)),
      ],
      out_specs=pl.BlockSpec((BR, OUT_DIM), lambda i: (i, 0)),
      out_shape=jax.ShapeDtypeStruct((N_NODES, OUT_DIM), jnp.float32),
  )(aggp2, ic, hr)


def kernel(x, edge_index, edge_label_index, W1_l, W1_r, b1, W2_l, W2_r, b2):
  i32 = jnp.int32
  f32 = jnp.float32
  src = edge_index[0].astype(i32)
  dst = edge_index[1].astype(i32)
  ls = edge_label_index[0].astype(i32)
  ld = edge_label_index[1].astype(i32)

  e1p = E1_PAD - N_EDGES
  src1 = jnp.concatenate([src, jnp.zeros((e1p,), i32)]).reshape(
      NS, E1_CPW, CHUNK)
  dst1 = jnp.concatenate([dst, jnp.full((e1p,), N_PAD - 1, i32)]).reshape(
      NS, E1_CPW, CHUNK)
  lp = L_PAD - N_LABEL
  ls2 = jnp.concatenate([ls, jnp.zeros((lp,), i32)]).reshape(NW, L_CPW, CHUNK)
  ld2 = jnp.concatenate([ld, jnp.zeros((lp,), i32)]).reshape(NW, L_CPW, CHUNK)

  xab = jnp.stack([
      x[:, :HALF_W],
      jnp.concatenate([x[:, HALF_W:], jnp.ones((N_NODES, 1), f32),
                       jnp.zeros((N_NODES, HALF_W - CNT_COL - 1), f32)],
                      axis=1),
  ])

  aggp1 = _segsum_l1(xab, src1, dst1)
  hw, hr, ic = _layer1(aggp1, x, W1_l, W1_r, b1.reshape(1, HID_DIM),
                       W2_l, W2_r, b2.reshape(1, OUT_DIM))
  aggp2 = _segsum_l2(hw, src1, dst1)
  z = _assemble_z(aggp2, ic, hr)
  out = _decode(z, ls2, ld2)
  return out[:N_LABEL]


# speedup vs baseline: 7.8735x; 1.0605x over previous
import functools

import jax
import jax.numpy as jnp
from jax import lax
from jax.experimental import pallas as pl
from jax.experimental.pallas import tpu as pltpu
from jax.experimental.pallas import tpu_sc as plsc

N_NODES = 10000
IN_DIM = 128
HID_DIM = 128
OUT_DIM = 64
N_EDGES = 320000
N_LABEL = 100000

NC, NS = 2, 16
NW = NC * NS
CHUNK = 128

N_PAD = 10240
ROWS_PER_SUB = N_PAD // NS

HALF_W = 72
CNT_COL = IN_DIM - HALF_W

E1_CPW = 157
E1_PAD = NS * E1_CPW * CHUNK

L_CPW = 25
L_PAD = NW * L_CPW * CHUNK

_MESH = plsc.VectorSubcoreMesh(core_axis_name="c", subcore_axis_name="s")
_SC_PARAMS = pltpu.CompilerParams(use_tc_tiling_on_sc=False,
                                  needs_layout_passes=False)


def _zero_rows(buf, width):
  z16 = jnp.zeros((16,), jnp.float32)
  ncol = (width + 15) // 16

  def zrow(r, carry):
    for c in range(ncol):
      buf[r, pl.ds(min(c * 16, width - 16), 16)] = z16
    return carry

  lax.fori_loop(0, CHUNK, zrow, 0)


def _make_segsum(width, cpw, split_cols, depth_g=2, depth_s=2):
  @functools.partial(
      pl.kernel,
      out_type=jax.ShapeDtypeStruct((NC, N_PAD, width), jnp.float32),
      mesh=_MESH,
      compiler_params=_SC_PARAMS,
      scratch_types=[
          pltpu.VMEM((cpw, CHUNK), jnp.int32),
          pltpu.VMEM((cpw, CHUNK), jnp.int32),
          pltpu.VMEM(((depth_g + depth_s) * CHUNK, width), jnp.float32),
          pltpu.VMEM_SHARED((N_PAD, width), jnp.float32),
          pltpu.SemaphoreType.DMA,
          pltpu.SemaphoreType.DMA,
      ],
  )
  def segsum(tab_hbm, src_hbm, dst_hbm, out_hbm,
             src_v, dst_v, rows2, acc_sh, gsem, ssem):
    cid = lax.axis_index("c")
    sid = lax.axis_index("s")
    r0 = sid * ROWS_PER_SUB
    tab = tab_hbm.at[cid] if split_cols else tab_hbm
    isel = sid if split_cols else cid * NS + sid

    _zero_rows(rows2, width)
    for t in range(ROWS_PER_SUB // CHUNK):
      pltpu.sync_copy(rows2.at[pl.ds(0, CHUNK)],
                      acc_sh.at[pl.ds(r0 + t * CHUNK, CHUNK)])
    plsc.subcore_barrier()
    pltpu.sync_copy(src_hbm.at[isel], src_v)
    pltpu.sync_copy(dst_hbm.at[isel], dst_v)

    nbuf = depth_g + depth_s

    def buf(v):
      return rows2.at[pl.ds((v % nbuf) * CHUNK, CHUNK)]

    def g_start(j, v):
      pltpu.async_copy(tab.at[src_v.at[j]], buf(v), gsem)

    def g_wait(v):
      pltpu.make_async_copy(tab.at[src_v.at[0]], buf(v), gsem).wait()

    def s_start(v):
      pltpu.async_copy(buf(v), acc_sh.at[dst_v.at[v]], ssem, add=True)

    def s_wait():
      pltpu.make_async_copy(buf(0), acc_sh.at[dst_v.at[0]], ssem).wait()

    def body(v, carry):
      u = v - depth_g

      @pl.when(v >= depth_g)
      def _process():
        g_wait(u)

        @pl.when(u >= depth_s)
        def _reclaim():
          s_wait()

        s_start(u)

      @pl.when(v < cpw)
      def _fetch():
        g_start(v, v)

      return carry

    lax.fori_loop(0, cpw + depth_g, body, 0)
    for _ in range(depth_s):
      s_wait()
    plsc.subcore_barrier()
    pltpu.sync_copy(acc_sh.at[pl.ds(r0, ROWS_PER_SUB)],
                    out_hbm.at[cid, pl.ds(r0, ROWS_PER_SUB)])

  return segsum


_segsum_l1 = _make_segsum(HALF_W, E1_CPW, split_cols=True)
_segsum_l2 = _make_segsum(OUT_DIM // 2, E1_CPW, split_cols=True,
                          depth_g=3, depth_s=2)


@functools.partial(
    pl.kernel,
    out_type=jax.ShapeDtypeStruct((L_PAD,), jnp.float32),
    mesh=_MESH,
    compiler_params=_SC_PARAMS,
    scratch_types=[
        pltpu.VMEM((L_CPW, CHUNK), jnp.int32),
        pltpu.VMEM((L_CPW, CHUNK), jnp.int32),
        pltpu.VMEM((4 * CHUNK, OUT_DIM), jnp.float32),
        pltpu.VMEM((4 * CHUNK, OUT_DIM), jnp.float32),
        pltpu.VMEM((L_CPW * CHUNK,), jnp.float32),
        pltpu.SemaphoreType.DMA,
        pltpu.SemaphoreType.DMA,
    ],
)
def _decode(z_hbm, ls_hbm, ld_hbm, out_hbm, ls_v, ld_v, zs2, zd2, out_v,
            ssm, dsm):
  cid = lax.axis_index("c")
  sid = lax.axis_index("s")
  wid = cid * NS + sid
  ib = wid * L_CPW
  pltpu.sync_copy(ls_hbm.at[wid], ls_v)
  pltpu.sync_copy(ld_hbm.at[wid], ld_v)

  def g_start(j, v):
    boff = (v % 4) * CHUNK
    pltpu.async_copy(z_hbm.at[ls_v.at[j]], zs2.at[pl.ds(boff, CHUNK)], ssm)
    pltpu.async_copy(z_hbm.at[ld_v.at[j]], zd2.at[pl.ds(boff, CHUNK)], dsm)

  def g_wait(v):
    boff = (v % 4) * CHUNK
    pltpu.make_async_copy(z_hbm.at[ls_v.at[0]], zs2.at[pl.ds(boff, CHUNK)],
                          ssm).wait()
    pltpu.make_async_copy(z_hbm.at[ld_v.at[0]], zd2.at[pl.ds(boff, CHUNK)],
                          dsm).wait()

  def visit(v, carry):
    u = v - 3

    @pl.when(v >= 3)
    def _process():
      g_wait(u)
      boff = (u % 4) * CHUNK
      lanes = lax.iota(jnp.int32, 16)

      def group(g, c2):
        r0 = boff + g * 16
        sv = jnp.zeros((16,), jnp.float32)
        for p in range(16):
          t = (zs2[r0 + p, pl.ds(0, 16)] * zd2[r0 + p, pl.ds(0, 16)])
          for c in range(1, OUT_DIM // 16):
            t = t + (zs2[r0 + p, pl.ds(c * 16, 16)] *
                     zd2[r0 + p, pl.ds(c * 16, 16)])
          s = jnp.sum(t)
          sv = jnp.where(lanes == p, s, sv)
        out_v[pl.ds(u * CHUNK + g * 16, 16)] = sv
        return c2

      lax.fori_loop(0, CHUNK // 16, group, 0)

    @pl.when(v < L_CPW)
    def _fetch():
      g_start(v, v)

    return carry

  lax.fori_loop(0, L_CPW + 3, visit, 0)
  pltpu.sync_copy(out_v, out_hbm.at[pl.ds(ib * CHUNK, L_CPW * CHUNK)])


def _layer1_body(aggp, xr, w1l, w1r, b1, w2l, w2r, b2, hw, hr, ic):
  a_lo = aggp[0]
  a_hi = aggp[1]
  inv = 1.0 / jnp.maximum(a_hi[:, CNT_COL], 1.0)
  m_lo = a_lo * inv[:, None]
  m_hi = a_hi[:, :CNT_COL] * inv[:, None]
  dn = (((1,), (1,)), ((), ()))
  f32 = jnp.float32
  h = (lax.dot_general(m_lo, w1l[:, :HALF_W], dn, preferred_element_type=f32)
       + lax.dot_general(m_hi, w1l[:, HALF_W:], dn, preferred_element_type=f32)
       + lax.dot_general(xr[...], w1r[...], dn, preferred_element_type=f32)
       + b1[...])
  h = jnp.maximum(h, 0.0)
  w2l_a = w2l[...]
  hw[0] = lax.dot_general(h, w2l_a[:OUT_DIM // 2], dn,
                          preferred_element_type=f32)
  hw[1] = lax.dot_general(h, w2l_a[OUT_DIM // 2:], dn,
                          preferred_element_type=f32)
  hr[...] = (lax.dot_general(h, w2r[...], dn, preferred_element_type=f32)
             + b2[...])
  ic[...] = inv[:, None]


def _layer1(aggp, x, W1_l, W1_r, b1, W2_l, W2_r, b2):
  BR = 1000
  return pl.pallas_call(
      lambda *refs: _layer1_body(refs[0][...], refs[1], refs[2][...],
                                 *refs[3:]),
      grid=(N_NODES // BR,),
      in_specs=[
          pl.BlockSpec((NC, BR, HALF_W), lambda i: (0, i, 0)),
          pl.BlockSpec((BR, IN_DIM), lambda i: (i, 0)),
          pl.BlockSpec((HID_DIM, IN_DIM), lambda i: (0, 0)),
          pl.BlockSpec((HID_DIM, IN_DIM), lambda i: (0, 0)),
          pl.BlockSpec((1, HID_DIM), lambda i: (0, 0)),
          pl.BlockSpec((OUT_DIM, HID_DIM), lambda i: (0, 0)),
          pl.BlockSpec((OUT_DIM, HID_DIM), lambda i: (0, 0)),
          pl.BlockSpec((1, OUT_DIM), lambda i: (0, 0)),
      ],
      out_specs=[
          pl.BlockSpec((NC, BR, OUT_DIM // 2), lambda i: (0, i, 0)),
          pl.BlockSpec((BR, OUT_DIM), lambda i: (i, 0)),
          pl.BlockSpec((BR, 1), lambda i: (i, 0)),
      ],
      out_shape=[
          jax.ShapeDtypeStruct((NC, N_NODES, OUT_DIM // 2), jnp.float32),
          jax.ShapeDtypeStruct((N_NODES, OUT_DIM), jnp.float32),
          jax.ShapeDtypeStruct((N_NODES, 1), jnp.float32),
      ],
  )(aggp, x, W1_l, W1_r, b1, W2_l, W2_r, b2)


def _assemble_body(aggp2, ic, hr, z):
  iv = ic[...]
  hra = hr[...]
  z[:, :OUT_DIM // 2] = aggp2[0] * iv + hra[:, :OUT_DIM // 2]
  z[:, OUT_DIM // 2:] = aggp2[1] * iv + hra[:, OUT_DIM // 2:]


def _assemble_z(aggp2, ic, hr):
  BR = 1000
  return pl.pallas_call(
      lambda *refs: _assemble_body(refs[0][...], *refs[1:]),
      grid=(N_NODES // BR,),
      in_specs=[
          pl.BlockSpec((NC, BR, OUT_DIM // 2), lambda i: (0, i, 0)),
          pl.BlockSpec((BR, 1), lambda i: (i, 0)),
          pl.BlockSpec((BR, OUT_DIM), lambda i: (i, 0)),
      ],
      out_specs=pl.BlockSpec((BR, OUT_DIM), lambda i: (i, 0)),
      out_shape=jax.ShapeDtypeStruct((N_NODES, OUT_DIM), jnp.float32),
  )(aggp2, ic, hr)


def kernel(x, edge_index, edge_label_index, W1_l, W1_r, b1, W2_l, W2_r, b2):
  i32 = jnp.int32
  f32 = jnp.float32
  src = edge_index[0].astype(i32)
  dst = edge_index[1].astype(i32)
  ls = edge_label_index[0].astype(i32)
  ld = edge_label_index[1].astype(i32)

  e1p = E1_PAD - N_EDGES
  src1 = jnp.concatenate([src, jnp.zeros((e1p,), i32)]).reshape(
      NS, E1_CPW, CHUNK)
  dst1 = jnp.concatenate([dst, jnp.full((e1p,), N_PAD - 1, i32)]).reshape(
      NS, E1_CPW, CHUNK)
  lp = L_PAD - N_LABEL
  ls2 = jnp.concatenate([ls, jnp.zeros((lp,), i32)]).reshape(NW, L_CPW, CHUNK)
  ld2 = jnp.concatenate([ld, jnp.zeros((lp,), i32)]).reshape(NW, L_CPW, CHUNK)

  xab = jnp.stack([
      x[:, :HALF_W],
      jnp.concatenate([x[:, HALF_W:], jnp.ones((N_NODES, 1), f32),
                       jnp.zeros((N_NODES, HALF_W - CNT_COL - 1), f32)],
                      axis=1),
  ])

  aggp1 = _segsum_l1(xab, src1, dst1)
  hw, hr, ic = _layer1(aggp1, x, W1_l, W1_r, b1.reshape(1, HID_DIM),
                       W2_l, W2_r, b2.reshape(1, OUT_DIM))
  aggp2 = _segsum_l2(hw, src1, dst1)
  z = _assemble_z(aggp2, ic, hr)
  out = _decode(z, ls2, ld2)
  return out[:N_LABEL]
